# baseline (device time: 14963 ns/iter reference)
import jax
import jax.numpy as jnp
from jax import lax
from jax.experimental import pallas as pl
from jax.experimental.pallas import tpu as pltpu

Q = 256
H = 128


def kernel(x):
    m_per, n = x.shape

    def body(x_ref, out_ref, mine_v, recv_v, send_sems, recv_sems,
             copy_sems):
        my_x = lax.axis_index("x")
        my_y = lax.axis_index("y")
        my_z = lax.axis_index("z")
        k = 2 * my_y + my_z
        ka = 3 - k
        kb = 2 * (1 - my_y) + my_z
        kc = 2 * my_y + (1 - my_z)
        base_mine = my_x * m_per
        base_rem = (1 - my_x) * m_per
        xp = (1 - my_x, my_y, my_z)
        yp = (my_x, 1 - my_y, my_z)
        zp = (my_x, my_y, 1 - my_z)

        barrier = pltpu.get_barrier_semaphore()
        for dev in (xp, yp, zp):
            pl.semaphore_signal(barrier, inc=1, device_id=dev,
                                device_id_type=pltpu.DeviceIdType.MESH)

        mine_v[...] = x_ref[...].astype(jnp.bfloat16)
        mine_out = pltpu.make_async_copy(
            mine_v, out_ref.at[pl.ds(base_mine, m_per), :], copy_sems.at[0])
        mine_out.start()

        pl.semaphore_wait(barrier, 3)

        def rcopy(src, rows, nrows, sem_i, dev):
            return pltpu.make_async_remote_copy(
                src_ref=src.at[pl.ds(rows, nrows), :],
                dst_ref=recv_v.at[pl.ds(rows, nrows), :],
                send_sem=send_sems.at[sem_i],
                recv_sem=recv_sems.at[sem_i],
                device_id=dev,
                device_id_type=pltpu.DeviceIdType.MESH,
            )

        a1 = rcopy(mine_v, k * Q, H, 0, xp)
        a2 = rcopy(mine_v, k * Q + H, H, 1, xp)
        a3 = rcopy(mine_v, ka * Q, Q, 2, xp)
        a1.start()
        a2.start()
        a3.start()

        def drain(rows, nrows, sem_i):
            cp = pltpu.make_async_copy(
                recv_v.at[pl.ds(rows, nrows), :],
                out_ref.at[pl.ds(base_rem + rows, nrows), :],
                copy_sems.at[sem_i],
            )
            cp.start()
            return cp

        a1.wait_recv()
        fy1 = rcopy(recv_v, k * Q, H, 3, yp)
        fz1 = rcopy(recv_v, k * Q, H, 4, zp)
        fy1.start()
        fz1.start()
        d1 = drain(k * Q, H, 1)
        a2.wait_recv()
        fy2 = rcopy(recv_v, k * Q + H, H, 5, yp)
        fz2 = rcopy(recv_v, k * Q + H, H, 6, zp)
        fy2.start()
        fz2.start()
        d2 = drain(k * Q + H, H, 2)

        fy1.wait_recv()
        d3 = drain(kb * Q, H, 3)
        fz1.wait_recv()
        d4 = drain(kc * Q, H, 4)
        fy2.wait_recv()
        d5 = drain(kb * Q + H, H, 5)
        fz2.wait_recv()
        d6 = drain(kc * Q + H, H, 6)
        a3.wait_recv()
        d7 = drain(ka * Q, Q, 7)

        mine_out.wait()
        for d in (d1, d2, d3, d4, d5, d6, d7):
            d.wait()
        a1.wait_send()
        a2.wait_send()
        a3.wait_send()
        fy1.wait_send()
        fz1.wait_send()
        fy2.wait_send()
        fz2.wait_send()

    return pl.pallas_call(
        body,
        out_shape=jax.ShapeDtypeStruct((2 * m_per, n), jnp.bfloat16),
        in_specs=[pl.BlockSpec(memory_space=pltpu.VMEM)],
        out_specs=pl.BlockSpec(memory_space=pl.ANY),
        scratch_shapes=[
            pltpu.VMEM((m_per, n), jnp.bfloat16),
            pltpu.VMEM((m_per, n), jnp.bfloat16),
            pltpu.SemaphoreType.DMA((7,)),
            pltpu.SemaphoreType.DMA((7,)),
            pltpu.SemaphoreType.DMA((8,)),
        ],
        compiler_params=pltpu.CompilerParams(collective_id=0),
    )(x)


# device time: 13971 ns/iter; 1.0710x vs baseline; 1.0710x over previous
import jax
import jax.numpy as jnp
from jax import lax
from jax.experimental import pallas as pl
from jax.experimental.pallas import tpu as pltpu

Q = 256
H = 64
NC = 4


def kernel(x):
    m_per, n = x.shape

    def body(x_ref, out_ref, send_sems, recv_sems):
        my_x = lax.axis_index("x")
        my_y = lax.axis_index("y")
        my_z = lax.axis_index("z")
        k = 2 * my_y + my_z
        ka = 3 - k
        base_mine = my_x * m_per
        base_rem = (1 - my_x) * m_per
        xp = (1 - my_x, my_y, my_z)
        yp = (my_x, 1 - my_y, my_z)
        zp = (my_x, my_y, 1 - my_z)

        barrier = pltpu.get_barrier_semaphore()
        for dev in (xp, yp, zp):
            pl.semaphore_signal(barrier, inc=1, device_id=dev,
                                device_id_type=pltpu.DeviceIdType.MESH)

        out_ref[pl.ds(base_mine, m_per), :] = x_ref[...].astype(jnp.bfloat16)

        pl.semaphore_wait(barrier, 3)

        def xcopy(rows, nrows, ssem, rsem, dev):
            return pltpu.make_async_remote_copy(
                src_ref=out_ref.at[pl.ds(rows, nrows), :],
                dst_ref=out_ref.at[pl.ds(rows, nrows), :],
                send_sem=send_sems.at[ssem],
                recv_sem=recv_sems.at[rsem],
                device_id=dev,
                device_id_type=pltpu.DeviceIdType.MESH,
            )

        achunks = [xcopy(base_mine + k * Q + c * H, H, c, c, xp)
                   for c in range(NC)]
        a3 = xcopy(base_mine + ka * Q, Q, NC, NC, xp)
        for a in achunks:
            a.start()
        a3.start()

        fwds = []
        for c in range(NC):
            achunks[c].wait_recv()
            fy = xcopy(base_rem + k * Q + c * H, H,
                       NC + 1 + 2 * c, NC + 1 + 2 * c, yp)
            fz = xcopy(base_rem + k * Q + c * H, H,
                       NC + 2 + 2 * c, NC + 2 + 2 * c, zp)
            fy.start()
            fz.start()
            fwds += [fy, fz]

        a3.wait_recv()
        for f in fwds:
            f.wait_recv()
        for a in achunks:
            a.wait_send()
        a3.wait_send()
        for f in fwds:
            f.wait_send()

    return pl.pallas_call(
        body,
        out_shape=jax.ShapeDtypeStruct((2 * m_per, n), jnp.bfloat16),
        in_specs=[pl.BlockSpec(memory_space=pltpu.VMEM)],
        out_specs=pl.BlockSpec(memory_space=pltpu.VMEM),
        scratch_shapes=[
            pltpu.SemaphoreType.DMA((3 * NC + 1,)),
            pltpu.SemaphoreType.DMA((3 * NC + 1,)),
        ],
        compiler_params=pltpu.CompilerParams(collective_id=0),
    )(x)
